# baseline (speedup 1.0000x reference)
import jax
import jax.numpy as jnp
from jax.experimental import pallas as pl
from jax.experimental.pallas import tpu as pltpu

DIM = 768
NUM_EXPERTS = 8
TOP_K = 2
NUM_TOKENS = 2048


def _moe_kernel(x_ref, wg_ref, we_ref, o_ref, xbf_ref, i1_ref, i2_ref, w1_ref, w2_ref):
    step = pl.program_id(0)

    @pl.when(step == 0)
    def _gate():
        x = x_ref[...]
        xbf_ref[...] = x.astype(jnp.bfloat16)
        logits = jnp.dot(x, wg_ref[...], preferred_element_type=jnp.float32)
        iota = jax.lax.broadcasted_iota(jnp.int32, logits.shape, 1)
        v1 = jnp.max(logits, axis=1, keepdims=True)
        i1 = jnp.min(jnp.where(logits == v1, iota, NUM_EXPERTS), axis=1, keepdims=True)
        l2 = jnp.where(iota == i1, -jnp.inf, logits)
        v2 = jnp.max(l2, axis=1, keepdims=True)
        i2 = jnp.min(jnp.where(l2 == v2, iota, NUM_EXPERTS), axis=1, keepdims=True)
        w1 = 1.0 / (1.0 + jnp.exp(v2 - v1))
        i1_ref[...] = i1
        i2_ref[...] = i2
        w1_ref[...] = w1
        w2_ref[...] = 1.0 - w1

    xbf = xbf_ref[...]
    i1 = i1_ref[...]
    i2 = i2_ref[...]
    w1 = w1_ref[...]
    w2 = w2_ref[...]

    def contrib(e, slot):
        scale = jnp.where(i1 == e, w1, 0.0) + jnp.where(i2 == e, w2, 0.0)
        y = jnp.dot(
            xbf, we_ref[slot].astype(jnp.bfloat16), preferred_element_type=jnp.float32
        )
        return scale * y

    ea = step * 4
    acc = (
        (contrib(ea, 0) + contrib(ea + 1, 1))
        + (contrib(ea + 2, 2) + contrib(ea + 3, 3))
    )

    @pl.when(step == 0)
    def _first():
        o_ref[...] = acc

    @pl.when(step != 0)
    def _rest():
        o_ref[...] += acc


def kernel(inputs, Wg, We):
    return pl.pallas_call(
        _moe_kernel,
        grid=(NUM_EXPERTS // 4,),
        in_specs=[
            pl.BlockSpec((NUM_TOKENS, DIM), lambda s: (0, 0)),
            pl.BlockSpec((DIM, NUM_EXPERTS), lambda s: (0, 0)),
            pl.BlockSpec((4, DIM, DIM), lambda s: (s, 0, 0)),
        ],
        out_specs=pl.BlockSpec((NUM_TOKENS, DIM), lambda s: (0, 0)),
        out_shape=jax.ShapeDtypeStruct((NUM_TOKENS, DIM), jnp.float32),
        scratch_shapes=[
            pltpu.VMEM((NUM_TOKENS, DIM), jnp.bfloat16),
            pltpu.VMEM((NUM_TOKENS, 1), jnp.int32),
            pltpu.VMEM((NUM_TOKENS, 1), jnp.int32),
            pltpu.VMEM((NUM_TOKENS, 1), jnp.float32),
            pltpu.VMEM((NUM_TOKENS, 1), jnp.float32),
        ],
    )(inputs, Wg, We)


# speedup vs baseline: 1.0270x; 1.0270x over previous
import jax
import jax.numpy as jnp
from jax.experimental import pallas as pl
from jax.experimental.pallas import tpu as pltpu

DIM = 768
NUM_EXPERTS = 8
TOP_K = 2
NUM_TOKENS = 2048


def _moe_kernel(x_ref, wg_ref, we_ref, o_ref, xbf_ref, i1_ref, i2_ref, w1_ref, w2_ref):
    step = pl.program_id(0)

    @pl.when(step == 0)
    def _gate():
        x = x_ref[...]
        xbf_ref[...] = x.astype(jnp.bfloat16)
        logits = jnp.dot(x, wg_ref[...], preferred_element_type=jnp.float32)
        iota = jax.lax.broadcasted_iota(jnp.int32, logits.shape, 1)
        v1 = jnp.max(logits, axis=1, keepdims=True)
        i1 = jnp.min(jnp.where(logits == v1, iota, NUM_EXPERTS), axis=1, keepdims=True)
        l2 = jnp.where(iota == i1, -jnp.inf, logits)
        v2 = jnp.max(l2, axis=1, keepdims=True)
        i2 = jnp.min(jnp.where(l2 == v2, iota, NUM_EXPERTS), axis=1, keepdims=True)
        w1 = 1.0 / (1.0 + jnp.exp(v2 - v1))
        i1_ref[...] = i1
        i2_ref[...] = i2
        w1_ref[...] = w1
        w2_ref[...] = 1.0 - w1

    xbf = xbf_ref[...]
    i1 = i1_ref[...]
    i2 = i2_ref[...]
    w1 = w1_ref[...]
    w2 = w2_ref[...]

    def contrib(e, slot):
        scale = jnp.where(i1 == e, w1, 0.0) + jnp.where(i2 == e, w2, 0.0)
        y = jnp.dot(
            xbf, we_ref[slot].astype(jnp.bfloat16), preferred_element_type=jnp.float32
        )
        return scale * y

    ea = step * 2
    acc = contrib(ea, 0) + contrib(ea + 1, 1)

    @pl.when(step == 0)
    def _first():
        o_ref[...] = acc

    @pl.when(step != 0)
    def _rest():
        o_ref[...] += acc


def kernel(inputs, Wg, We):
    return pl.pallas_call(
        _moe_kernel,
        grid=(NUM_EXPERTS // 2,),
        in_specs=[
            pl.BlockSpec((NUM_TOKENS, DIM), lambda s: (0, 0)),
            pl.BlockSpec((DIM, NUM_EXPERTS), lambda s: (0, 0)),
            pl.BlockSpec((2, DIM, DIM), lambda s: (s, 0, 0)),
        ],
        out_specs=pl.BlockSpec((NUM_TOKENS, DIM), lambda s: (0, 0)),
        out_shape=jax.ShapeDtypeStruct((NUM_TOKENS, DIM), jnp.float32),
        scratch_shapes=[
            pltpu.VMEM((NUM_TOKENS, DIM), jnp.bfloat16),
            pltpu.VMEM((NUM_TOKENS, 1), jnp.int32),
            pltpu.VMEM((NUM_TOKENS, 1), jnp.int32),
            pltpu.VMEM((NUM_TOKENS, 1), jnp.float32),
            pltpu.VMEM((NUM_TOKENS, 1), jnp.float32),
        ],
    )(inputs, Wg, We)
